# baseline (speedup 1.0000x reference)
import functools

import jax
import jax.numpy as jnp
from jax import lax
from jax.experimental import pallas as pl
from jax.experimental.pallas import tpu as pltpu
from jax.experimental.pallas import tpu_sc as plsc

B = 16384
SED = 32
PED = 64
NUM_MODES = 12


def _sc_gather(table2, idx2):
    NW, b_per_w = idx2.shape
    mesh = plsc.VectorSubcoreMesh(core_axis_name="c", subcore_axis_name="s")
    nc = mesh.num_cores

    @functools.partial(
        pl.kernel,
        out_type=jax.ShapeDtypeStruct((NW * b_per_w, 2 * SED), jnp.float32),
        mesh=mesh,
        scratch_types=[
            pltpu.VMEM((b_per_w,), jnp.int32),
            pltpu.VMEM((b_per_w, 2 * SED), jnp.float32),
            pltpu.SemaphoreType.DMA,
        ],
    )
    def gather_kernel(table_hbm, idx_hbm, out_hbm, idx_v, rows_v, sem):
        wid = lax.axis_index("s") * nc + lax.axis_index("c")
        base = wid * b_per_w
        pltpu.sync_copy(idx_hbm.at[wid], idx_v)

        def body(g, carry):
            v = idx_v[pl.ds(g * 16, 16)]
            for l in range(16):
                r = v[l]
                pltpu.async_copy(
                    table_hbm.at[pl.ds(r, 1)],
                    rows_v.at[pl.ds(g * 16 + l, 1)],
                    sem,
                )
            return carry

        lax.fori_loop(0, b_per_w // 16, body, 0)
        pltpu.make_async_copy(
            table_hbm.at[pl.ds(0, b_per_w)], rows_v, sem
        ).wait()
        pltpu.sync_copy(rows_v, out_hbm.at[pl.ds(base, b_per_w)])

    return gather_kernel(table2, idx2)


def _tc_fused_t(rows, uid2d, tmT, tsT, mode_table, W_time, b_time2d, WpT, b_pref2d):
    bs = 4096
    grid = (B // bs,)

    def body(u_ref, uid_ref, tm_ref, ts_ref, mt_ref, wt_ref, bt_ref, wpt_ref,
             bp_ref, o_ref):
        u2 = u_ref[...]
        off = uid_ref[...] & 1
        u = jnp.where(off == 0, u2[:, 0:SED], 0.0)
        u = u + jnp.where(off == 1, u2[:, SED:], 0.0)
        ts = ts_ref[...]
        tm = tm_ref[...]
        wpt = wpt_ref[...]
        time_embT = lax.dot_general(
            wt_ref[...], ts, (((0,), (0,)), ((), ())),
            preferred_element_type=jnp.float32,
        ) + bt_ref[...]
        onehotT = (
            lax.broadcasted_iota(jnp.int32, (NUM_MODES, bs), 0) == tm
        ).astype(jnp.float32)
        mode_embT = lax.dot_general(
            mt_ref[...], onehotT, (((0,), (0,)), ((), ())),
            preferred_element_type=jnp.float32,
        )
        acc = lax.dot_general(
            wpt[:, 0:SED], u, (((1,), (1,)), ((), ())),
            preferred_element_type=jnp.float32,
        )
        acc = acc + lax.dot_general(
            wpt[:, SED : 2 * SED], mode_embT, (((1,), (0,)), ((), ())),
            preferred_element_type=jnp.float32,
        )
        acc = acc + lax.dot_general(
            wpt[:, 2 * SED :], time_embT, (((1,), (0,)), ((), ())),
            preferred_element_type=jnp.float32,
        )
        o_ref[...] = acc + bp_ref[...]

    return pl.pallas_call(
        body,
        grid=grid,
        in_specs=[
            pl.BlockSpec((bs, 2 * SED), lambda i: (i, 0)),
            pl.BlockSpec((bs, 1), lambda i: (i, 0)),
            pl.BlockSpec((1, bs), lambda i: (0, i)),
            pl.BlockSpec((6, bs), lambda i: (0, i)),
            pl.BlockSpec((NUM_MODES, SED), lambda i: (0, 0)),
            pl.BlockSpec((6, SED), lambda i: (0, 0)),
            pl.BlockSpec((SED, 1), lambda i: (0, 0)),
            pl.BlockSpec((PED, 3 * SED), lambda i: (0, 0)),
            pl.BlockSpec((PED, 1), lambda i: (0, 0)),
        ],
        out_specs=pl.BlockSpec((PED, bs), lambda i: (0, i)),
        out_shape=jax.ShapeDtypeStruct((PED, B), jnp.float32),
    )(rows, uid2d, tmT, tsT, mode_table, W_time, b_time2d, WpT, b_pref2d)


def kernel(user_id, transport_mode, timestamp, user_table, mode_table,
           W_time, b_time, W_pref, b_pref):
    info = plsc.get_sparse_core_info()
    NW = info.num_cores * info.num_subcores
    uid = user_id.astype(jnp.int32)
    idx2 = (uid >> 1).reshape(NW, B // NW)
    rows = _sc_gather(user_table.reshape(-1, 2 * SED), idx2)
    outT = _tc_fused_t(
        rows,
        uid.reshape(B, 1),
        transport_mode.astype(jnp.int32).reshape(1, B),
        timestamp.T,
        mode_table,
        W_time,
        b_time.reshape(SED, 1),
        W_pref.T,
        b_pref.reshape(PED, 1),
    )
    return outT.T


# speedup vs baseline: 1.8379x; 1.8379x over previous
import functools

import jax
import jax.numpy as jnp
from jax import lax
from jax.experimental import pallas as pl
from jax.experimental.pallas import tpu as pltpu
from jax.experimental.pallas import tpu_sc as plsc

B = 16384
SED = 32
PED = 64
NUM_MODES = 12


def _sc_gather(user_table, idx2):
    NW, b_per_w = idx2.shape
    mesh = plsc.VectorSubcoreMesh(core_axis_name="c", subcore_axis_name="s")
    nc = mesh.num_cores

    @functools.partial(
        pl.kernel,
        out_type=jax.ShapeDtypeStruct((NW * b_per_w, SED), jnp.float32),
        mesh=mesh,
        scratch_types=[
            pltpu.VMEM((b_per_w,), jnp.int32),
            pltpu.VMEM((b_per_w, SED), jnp.float32),
            pltpu.SemaphoreType.DMA,
        ],
    )
    def gather_kernel(table_hbm, idx_hbm, out_hbm, idx_v, rows_v, sem):
        wid = lax.axis_index("s") * nc + lax.axis_index("c")
        base = wid * b_per_w
        pltpu.sync_copy(idx_hbm.at[wid], idx_v)

        def body(g, carry):
            v = idx_v[pl.ds(g * 16, 16)]
            for l in range(16):
                r = v[l]
                pltpu.async_copy(
                    table_hbm.at[pl.ds(r, 1)],
                    rows_v.at[pl.ds(g * 16 + l, 1)],
                    sem,
                )
            return carry

        lax.fori_loop(0, b_per_w // 16, body, 0)
        pltpu.make_async_copy(
            table_hbm.at[pl.ds(0, b_per_w)], rows_v, sem
        ).wait()
        pltpu.sync_copy(rows_v, out_hbm.at[pl.ds(base, b_per_w)])

    return gather_kernel(user_table, idx2)


def _tc_fused_t(rows, tmT, tsT, mode_table, W_time, b_time2d, WpT, b_pref2d):
    bs = 4096
    grid = (B // bs,)

    def body(u_ref, tm_ref, ts_ref, mt_ref, wt_ref, bt_ref, wpt_ref, bp_ref,
             o_ref):
        u = u_ref[...]
        ts = ts_ref[...]
        tm = tm_ref[...]
        wpt = wpt_ref[...]
        time_embT = lax.dot_general(
            wt_ref[...], ts, (((0,), (0,)), ((), ())),
            preferred_element_type=jnp.float32,
        ) + bt_ref[...]
        onehotT = (
            lax.broadcasted_iota(jnp.int32, (NUM_MODES, bs), 0) == tm
        ).astype(jnp.float32)
        mode_embT = lax.dot_general(
            mt_ref[...], onehotT, (((0,), (0,)), ((), ())),
            preferred_element_type=jnp.float32,
        )
        acc = lax.dot_general(
            wpt[:, 0:SED], u, (((1,), (1,)), ((), ())),
            preferred_element_type=jnp.float32,
        )
        acc = acc + lax.dot_general(
            wpt[:, SED : 2 * SED], mode_embT, (((1,), (0,)), ((), ())),
            preferred_element_type=jnp.float32,
        )
        acc = acc + lax.dot_general(
            wpt[:, 2 * SED :], time_embT, (((1,), (0,)), ((), ())),
            preferred_element_type=jnp.float32,
        )
        o_ref[...] = acc + bp_ref[...]

    return pl.pallas_call(
        body,
        grid=grid,
        in_specs=[
            pl.BlockSpec((bs, SED), lambda i: (i, 0)),
            pl.BlockSpec((1, bs), lambda i: (0, i)),
            pl.BlockSpec((6, bs), lambda i: (0, i)),
            pl.BlockSpec((NUM_MODES, SED), lambda i: (0, 0)),
            pl.BlockSpec((6, SED), lambda i: (0, 0)),
            pl.BlockSpec((SED, 1), lambda i: (0, 0)),
            pl.BlockSpec((PED, 3 * SED), lambda i: (0, 0)),
            pl.BlockSpec((PED, 1), lambda i: (0, 0)),
        ],
        out_specs=pl.BlockSpec((PED, bs), lambda i: (0, i)),
        out_shape=jax.ShapeDtypeStruct((PED, B), jnp.float32),
    )(rows, tmT, tsT, mode_table, W_time, b_time2d, WpT, b_pref2d)


def kernel(user_id, transport_mode, timestamp, user_table, mode_table,
           W_time, b_time, W_pref, b_pref):
    info = plsc.get_sparse_core_info()
    NW = info.num_cores * info.num_subcores
    uid = user_id.astype(jnp.int32)
    idx2 = uid.reshape(NW, B // NW)
    rows = _sc_gather(user_table, idx2)
    outT = _tc_fused_t(
        rows,
        transport_mode.astype(jnp.int32).reshape(1, B),
        timestamp.T,
        mode_table,
        W_time,
        b_time.reshape(SED, 1),
        W_pref.T,
        b_pref.reshape(PED, 1),
    )
    return outT.T
